# baseline (speedup 1.0000x reference)
import functools

import jax
import jax.numpy as jnp
from jax import lax
from jax.experimental import pallas as pl
from jax.experimental.pallas import tpu as pltpu
from jax.experimental.pallas import tpu_sc as plsc

N = 10000
E = 320000
D = 128
DH = D // 2

NC = 2
NS = 16
EPT = E // NS
C = 80
J = EPT // C
RPT = N // NS
RCHS = (80, 80, 80, 80, 80, 80, 80, 65)

NBUF = 5
NGIF = 3


def _mm_body(x_ref, w_ref, o_ref):
    h = jnp.dot(x_ref[...], w_ref[...], preferred_element_type=jnp.float32)
    o_ref[0] = h[:, :DH]
    o_ref[1] = h[:, DH:]


def _matmul_split(x, W):
    blk = 10000
    grid = N // blk
    return pl.pallas_call(
        _mm_body,
        grid=(grid,),
        in_specs=[
            pl.BlockSpec((blk, D), lambda i: (i, 0)),
            pl.BlockSpec((D, D), lambda i: (0, 0)),
        ],
        out_specs=pl.BlockSpec((2, blk, DH), lambda i: (0, i, 0)),
        out_shape=jax.ShapeDtypeStruct((2, N, DH), jnp.float32),
    )(x, W)


def _lane_splat(vec, e):
    idx = jnp.full((16, 1), e, jnp.int32)
    return lax.gather(
        vec, idx,
        dimension_numbers=lax.GatherDimensionNumbers(
            offset_dims=(), collapsed_slice_dims=(0,), start_index_map=(0,)),
        slice_sizes=(1,),
        mode=lax.GatherScatterMode.PROMISE_IN_BOUNDS)


def _sc_body(h_hbm, src_hbm, dst_hbm, vals_hbm, b_hbm, out_hbm,
             src_v, dst_v, vals_v, rows0, rows1, rows2, rows3, rows4,
             b_v, acc_sh, sem_g, sem_s):
    c = lax.axis_index("c")
    s = lax.axis_index("s")
    rows = [rows0, rows1, rows2, rows3, rows4]

    cp1 = pltpu.async_copy(src_hbm.at[s], src_v, sem_g.at[0])
    cp2 = pltpu.async_copy(dst_hbm.at[s], dst_v, sem_g.at[1])
    cp3 = pltpu.async_copy(vals_hbm.at[s], vals_v, sem_g.at[2])
    cp4 = pltpu.async_copy(b_hbm, b_v, sem_g.at[3])

    @plsc.parallel_loop(0, C)
    def _z(r):
        for d in range(DH // 16):
            rows0[r, pl.ds(d * 16, 16)] = jnp.zeros((16,), jnp.float32)
    row0 = s * RPT
    for rch in RCHS:
        pltpu.sync_copy(rows0.at[pl.ds(0, rch)],
                        acc_sh.at[pl.ds(row0, rch)])
        row0 += rch
    cp1.wait()
    cp2.wait()
    cp3.wait()
    cp4.wait()
    plsc.subcore_barrier()

    h_half = h_hbm.at[c]

    def start_gather(j, b):
        pltpu.async_copy(h_half.at[src_v.at[j]], rows[b], sem_g.at[b])

    def wait_gather(b):
        pltpu.make_async_copy(h_half.at[src_v.at[0]], rows[b],
                              sem_g.at[b]).wait()

    def start_scatter(j, b):
        pltpu.async_copy(rows[b], acc_sh.at[dst_v.at[j]], sem_s.at[b],
                         add=True)

    def wait_scatter(b):
        pltpu.make_async_copy(rows[b], acc_sh.at[dst_v.at[0]],
                              sem_s.at[b]).wait()

    def scale(j, b):
        @plsc.parallel_loop(0, C // 16, unroll=5)
        def _grp(g):
            vgrp = vals_v[j, pl.ds(g * 16, 16)]
            for e in range(16):
                vsplat = _lane_splat(vgrp, e)
                row = g * 16 + e
                for d in range(DH // 16):
                    sl = pl.ds(d * 16, 16)
                    rows[b][row, sl] = rows[b][row, sl] * vsplat

    for b in range(NGIF):
        start_gather(b, b)

    @pl.loop(0, J // NBUF)
    def _iter(k):
        for b in range(NBUF):
            j = k * NBUF + b
            jn = j + NGIF
            bn = (b + NGIF) % NBUF

            @pl.when(jn < J)
            def _():
                @pl.when(jn >= NBUF)
                def _():
                    wait_scatter(bn)
                start_gather(jn, bn)

            wait_gather(b)
            scale(j, b)
            start_scatter(j, b)

    for b in range(NBUF):
        wait_scatter(b)

    plsc.subcore_barrier()

    bvec = [b_v[pl.ds(c * DH + d * 16, 16)] for d in range(DH // 16)]
    row0 = s * RPT
    for rch in RCHS:
        pltpu.sync_copy(acc_sh.at[pl.ds(row0, rch)], rows0.at[pl.ds(0, rch)])

        @plsc.parallel_loop(0, rch)
        def _relu(r):
            for d in range(DH // 16):
                sl = pl.ds(d * 16, 16)
                rows0[r, sl] = jnp.maximum(rows0[r, sl] + bvec[d], 0.0)

        pltpu.sync_copy(rows0.at[pl.ds(0, rch)],
                        out_hbm.at[pl.ds(row0, rch), pl.ds(c * DH, DH)])
        row0 += rch


_sc_kernel = functools.partial(
    pl.kernel,
    out_type=jax.ShapeDtypeStruct((N, D), jnp.float32),
    mesh=plsc.VectorSubcoreMesh(core_axis_name="c", subcore_axis_name="s",
                                num_cores=NC, num_subcores=NS),
    compiler_params=pltpu.CompilerParams(use_tc_tiling_on_sc=False),
    scratch_types=[
        pltpu.VMEM((J, C), jnp.int32),
        pltpu.VMEM((J, C), jnp.int32),
        pltpu.VMEM((J, C), jnp.float32),
        pltpu.VMEM((C, DH), jnp.float32),
        pltpu.VMEM((C, DH), jnp.float32),
        pltpu.VMEM((C, DH), jnp.float32),
        pltpu.VMEM((C, DH), jnp.float32),
        pltpu.VMEM((C, DH), jnp.float32),
        pltpu.VMEM((D,), jnp.float32),
        pltpu.VMEM_SHARED((N, DH), jnp.float32),
        pltpu.SemaphoreType.DMA((NBUF,)),
        pltpu.SemaphoreType.DMA((NBUF,)),
    ],
)(_sc_body)


@jax.jit
def kernel(x, edge_index, edge_vals, W, b):
    h2 = _matmul_split(x, W)
    src = edge_index[0].reshape(NS, J, C)
    dst = edge_index[1].reshape(NS, J, C)
    vals = edge_vals.reshape(NS, J, C)
    return _sc_kernel(h2, src, dst, vals, b)


# speedup vs baseline: 1.1816x; 1.1816x over previous
import functools

import jax
import jax.numpy as jnp
from jax import lax
from jax.experimental import pallas as pl
from jax.experimental.pallas import tpu as pltpu
from jax.experimental.pallas import tpu_sc as plsc

N = 10000
E = 320000
D = 128
DH = D // 2

NC = 2
NS = 16
EPT = E // NS
C = 80
J = EPT // C
RPT = N // NS
RCHS = (80, 80, 80, 80, 80, 80, 80, 65)

NBUF = 5
NGIF = 3


def _mm_body(x_ref, w_ref, o_ref):
    h = jnp.dot(x_ref[...], w_ref[...], preferred_element_type=jnp.float32)
    o_ref[0] = h[:, :DH]
    o_ref[1] = h[:, DH:]


def _matmul_split(x, W):
    blk = 10000
    grid = N // blk
    return pl.pallas_call(
        _mm_body,
        grid=(grid,),
        in_specs=[
            pl.BlockSpec((blk, D), lambda i: (i, 0)),
            pl.BlockSpec((D, D), lambda i: (0, 0)),
        ],
        out_specs=pl.BlockSpec((2, blk, DH), lambda i: (0, i, 0)),
        out_shape=jax.ShapeDtypeStruct((2, N, DH), jnp.float32),
    )(x, W)


def _lane_splat(vec, e):
    idx = jnp.full((16, 1), e, jnp.int32)
    return lax.gather(
        vec, idx,
        dimension_numbers=lax.GatherDimensionNumbers(
            offset_dims=(), collapsed_slice_dims=(0,), start_index_map=(0,)),
        slice_sizes=(1,),
        mode=lax.GatherScatterMode.PROMISE_IN_BOUNDS)


def _sc_body(h_hbm, src_hbm, dst_hbm, vals_hbm, b_hbm, out_hbm,
             src_v, dst_v, vals_v, rows0, rows1, rows2, rows3, rows4,
             b_v, acc_sh, sem_g, sem_s):
    c = lax.axis_index("c")
    s = lax.axis_index("s")
    rows = [rows0, rows1, rows2, rows3, rows4]

    cp1 = pltpu.async_copy(src_hbm.at[s], src_v, sem_g.at[0])
    cp2 = pltpu.async_copy(dst_hbm.at[s], dst_v, sem_g.at[1])
    cp3 = pltpu.async_copy(vals_hbm.at[s], vals_v, sem_g.at[2])
    cp4 = pltpu.async_copy(b_hbm, b_v, sem_g.at[3])

    @plsc.parallel_loop(0, C)
    def _z(r):
        for d in range(DH // 16):
            rows0[r, pl.ds(d * 16, 16)] = jnp.zeros((16,), jnp.float32)
    row0 = s * RPT
    for rch in RCHS:
        pltpu.sync_copy(rows0.at[pl.ds(0, rch)],
                        acc_sh.at[pl.ds(row0, rch)])
        row0 += rch
    cp1.wait()
    cp2.wait()
    cp3.wait()
    cp4.wait()
    plsc.subcore_barrier()

    h_half = h_hbm.at[c]

    def start_gather(j, b):
        pltpu.async_copy(h_half.at[src_v.at[j]], rows[b], sem_g.at[b])

    def wait_gather(b):
        pltpu.make_async_copy(h_half.at[src_v.at[0]], rows[b],
                              sem_g.at[b]).wait()

    def start_scatter(j, b):
        pltpu.async_copy(rows[b], acc_sh.at[dst_v.at[j]], sem_s.at[b],
                         add=True)

    def wait_scatter(b):
        pltpu.make_async_copy(rows[b], acc_sh.at[dst_v.at[0]],
                              sem_s.at[b]).wait()

    def scale(j, b):
        @plsc.parallel_loop(0, C // 16)
        def _grp(g):
            vgrp = vals_v[j, pl.ds(g * 16, 16)]
            for e in range(16):
                vsplat = _lane_splat(vgrp, e)
                row = g * 16 + e
                for d in range(DH // 16):
                    sl = pl.ds(d * 16, 16)
                    rows[b][row, sl] = rows[b][row, sl] * vsplat

    for b in range(NGIF):
        start_gather(b, b)

    @pl.loop(0, J // NBUF)
    def _iter(k):
        for b in range(NBUF):
            j = k * NBUF + b
            jn = j + NGIF
            bn = (b + NGIF) % NBUF

            @pl.when(jn < J)
            def _():
                @pl.when(jn >= NBUF)
                def _():
                    wait_scatter(bn)
                start_gather(jn, bn)

            wait_gather(b)
            scale(j, b)
            start_scatter(j, b)

    for b in range(NBUF):
        wait_scatter(b)

    plsc.subcore_barrier()

    bvec = [b_v[pl.ds(c * DH + d * 16, 16)] for d in range(DH // 16)]
    row0 = s * RPT
    for rch in RCHS:
        pltpu.sync_copy(acc_sh.at[pl.ds(row0, rch)], rows0.at[pl.ds(0, rch)])

        @plsc.parallel_loop(0, rch)
        def _relu(r):
            for d in range(DH // 16):
                sl = pl.ds(d * 16, 16)
                rows0[r, sl] = jnp.maximum(rows0[r, sl] + bvec[d], 0.0)

        pltpu.sync_copy(rows0.at[pl.ds(0, rch)],
                        out_hbm.at[pl.ds(row0, rch), pl.ds(c * DH, DH)])
        row0 += rch


_sc_kernel = functools.partial(
    pl.kernel,
    out_type=jax.ShapeDtypeStruct((N, D), jnp.float32),
    mesh=plsc.VectorSubcoreMesh(core_axis_name="c", subcore_axis_name="s",
                                num_cores=NC, num_subcores=NS),
    compiler_params=pltpu.CompilerParams(use_tc_tiling_on_sc=False),
    scratch_types=[
        pltpu.VMEM((J, C), jnp.int32),
        pltpu.VMEM((J, C), jnp.int32),
        pltpu.VMEM((J, C), jnp.float32),
        pltpu.VMEM((C, DH), jnp.float32),
        pltpu.VMEM((C, DH), jnp.float32),
        pltpu.VMEM((C, DH), jnp.float32),
        pltpu.VMEM((C, DH), jnp.float32),
        pltpu.VMEM((C, DH), jnp.float32),
        pltpu.VMEM((D,), jnp.float32),
        pltpu.VMEM_SHARED((N, DH), jnp.float32),
        pltpu.SemaphoreType.DMA((NBUF,)),
        pltpu.SemaphoreType.DMA((NBUF,)),
    ],
)(_sc_body)


@jax.jit
def kernel(x, edge_index, edge_vals, W, b):
    h2 = _matmul_split(x, W)
    src = edge_index[0].reshape(NS, J, C)
    dst = edge_index[1].reshape(NS, J, C)
    vals = edge_vals.reshape(NS, J, C)
    return _sc_kernel(h2, src, dst, vals, b)


# speedup vs baseline: 1.2008x; 1.0163x over previous
import functools

import jax
import jax.numpy as jnp
from jax import lax
from jax.experimental import pallas as pl
from jax.experimental.pallas import tpu as pltpu
from jax.experimental.pallas import tpu_sc as plsc

N = 10000
E = 320000
D = 128
DH = D // 2

NC = 2
NS = 16
EPT = E // NS
C = 80
J = EPT // C
RPT = N // NS
RCHS = (80, 80, 80, 80, 80, 80, 80, 65)

NBUF = 5
NGIF = 3


def _mm_body(x_ref, w_ref, o_ref):
    h = jnp.dot(x_ref[...], w_ref[...], preferred_element_type=jnp.float32)
    o_ref[0] = h[:, :DH]
    o_ref[1] = h[:, DH:]


def _matmul_split(x, W):
    blk = 10000
    grid = N // blk
    return pl.pallas_call(
        _mm_body,
        grid=(grid,),
        in_specs=[
            pl.BlockSpec((blk, D), lambda i: (i, 0)),
            pl.BlockSpec((D, D), lambda i: (0, 0)),
        ],
        out_specs=pl.BlockSpec((2, blk, DH), lambda i: (0, i, 0)),
        out_shape=jax.ShapeDtypeStruct((2, N, DH), jnp.float32),
    )(x, W)


def _lane_splat(vec, e):
    idx = jnp.full((16, 1), e, jnp.int32)
    return lax.gather(
        vec, idx,
        dimension_numbers=lax.GatherDimensionNumbers(
            offset_dims=(), collapsed_slice_dims=(0,), start_index_map=(0,)),
        slice_sizes=(1,),
        mode=lax.GatherScatterMode.PROMISE_IN_BOUNDS)


def _sc_body(h_hbm, src_hbm, dst_hbm, vals_hbm, b_hbm, out_hbm,
             src_v, dst_v, vals_v, rows0, rows1, rows2, rows3, rows4,
             b_v, acc_sh, sem_g, sem_s):
    c = lax.axis_index("c")
    s = lax.axis_index("s")
    rows = [rows0, rows1, rows2, rows3, rows4]

    cp1 = pltpu.async_copy(src_hbm.at[s], src_v, sem_g.at[0])
    cp2 = pltpu.async_copy(dst_hbm.at[s], dst_v, sem_g.at[1])
    cp3 = pltpu.async_copy(vals_hbm.at[s], vals_v, sem_g.at[2])
    cp4 = pltpu.async_copy(b_hbm, b_v, sem_g.at[3])

    @plsc.parallel_loop(0, C)
    def _z(r):
        for d in range(DH // 16):
            rows0[r, pl.ds(d * 16, 16)] = jnp.zeros((16,), jnp.float32)
    row0 = s * RPT
    zcps = []
    for rch in RCHS:
        zcps.append(pltpu.async_copy(rows0.at[pl.ds(0, rch)],
                                     acc_sh.at[pl.ds(row0, rch)],
                                     sem_s.at[4]))
        row0 += rch
    for zcp in zcps:
        zcp.wait()
    cp1.wait()
    cp2.wait()
    cp3.wait()
    cp4.wait()
    plsc.subcore_barrier()

    h_half = h_hbm.at[c]

    def start_gather(j, b):
        pltpu.async_copy(h_half.at[src_v.at[j]], rows[b], sem_g.at[b])

    def wait_gather(b):
        pltpu.make_async_copy(h_half.at[src_v.at[0]], rows[b],
                              sem_g.at[b]).wait()

    def start_scatter(j, b):
        pltpu.async_copy(rows[b], acc_sh.at[dst_v.at[j]], sem_s.at[b],
                         add=True)

    def wait_scatter(b):
        pltpu.make_async_copy(rows[b], acc_sh.at[dst_v.at[0]],
                              sem_s.at[b]).wait()

    def scale(j, b):
        @plsc.parallel_loop(0, C // 16)
        def _grp(g):
            vgrp = vals_v[j, pl.ds(g * 16, 16)]
            for e in range(16):
                vsplat = _lane_splat(vgrp, e)
                row = g * 16 + e
                for d in range(DH // 16):
                    sl = pl.ds(d * 16, 16)
                    rows[b][row, sl] = rows[b][row, sl] * vsplat

    for b in range(NGIF):
        start_gather(b, b)

    @pl.loop(0, J // NBUF)
    def _iter(k):
        for b in range(NBUF):
            j = k * NBUF + b
            jn = j + NGIF
            bn = (b + NGIF) % NBUF

            @pl.when(jn < J)
            def _():
                @pl.when(jn >= NBUF)
                def _():
                    wait_scatter(bn)
                start_gather(jn, bn)

            wait_gather(b)
            scale(j, b)
            start_scatter(j, b)

    for b in range(NBUF):
        wait_scatter(b)

    plsc.subcore_barrier()

    bvec = [b_v[pl.ds(c * DH + d * 16, 16)] for d in range(DH // 16)]
    row0s, r0 = [], s * RPT
    for rch in RCHS:
        row0s.append(r0)
        r0 += rch

    def ep_load(i, b):
        pltpu.async_copy(acc_sh.at[pl.ds(row0s[i], RCHS[i])],
                         rows[b].at[pl.ds(0, RCHS[i])], sem_g.at[b])

    ep_load(0, 0)
    for i, rch in enumerate(RCHS):
        b = i % 2
        buf = rows[b]
        pltpu.make_async_copy(acc_sh.at[pl.ds(row0s[i], rch)],
                              buf.at[pl.ds(0, rch)], sem_g.at[b]).wait()
        if i + 1 < len(RCHS):
            nb = (i + 1) % 2
            if i >= 1:
                pltpu.make_async_copy(
                    rows[nb].at[pl.ds(0, RCHS[i - 1])],
                    out_hbm.at[pl.ds(row0s[i - 1], RCHS[i - 1]),
                               pl.ds(c * DH, DH)],
                    sem_s.at[nb]).wait()
            ep_load(i + 1, nb)

        @plsc.parallel_loop(0, rch)
        def _relu(r):
            for d in range(DH // 16):
                sl = pl.ds(d * 16, 16)
                buf[r, sl] = jnp.maximum(buf[r, sl] + bvec[d], 0.0)

        pltpu.async_copy(buf.at[pl.ds(0, rch)],
                         out_hbm.at[pl.ds(row0s[i], rch), pl.ds(c * DH, DH)],
                         sem_s.at[b])
    for i in (len(RCHS) - 2, len(RCHS) - 1):
        b = i % 2
        pltpu.make_async_copy(rows[b].at[pl.ds(0, RCHS[i])],
                              out_hbm.at[pl.ds(row0s[i], RCHS[i]),
                                         pl.ds(c * DH, DH)],
                              sem_s.at[b]).wait()


_sc_kernel = functools.partial(
    pl.kernel,
    out_type=jax.ShapeDtypeStruct((N, D), jnp.float32),
    mesh=plsc.VectorSubcoreMesh(core_axis_name="c", subcore_axis_name="s",
                                num_cores=NC, num_subcores=NS),
    compiler_params=pltpu.CompilerParams(use_tc_tiling_on_sc=False),
    scratch_types=[
        pltpu.VMEM((J, C), jnp.int32),
        pltpu.VMEM((J, C), jnp.int32),
        pltpu.VMEM((J, C), jnp.float32),
        pltpu.VMEM((C, DH), jnp.float32),
        pltpu.VMEM((C, DH), jnp.float32),
        pltpu.VMEM((C, DH), jnp.float32),
        pltpu.VMEM((C, DH), jnp.float32),
        pltpu.VMEM((C, DH), jnp.float32),
        pltpu.VMEM((D,), jnp.float32),
        pltpu.VMEM_SHARED((N, DH), jnp.float32),
        pltpu.SemaphoreType.DMA((NBUF,)),
        pltpu.SemaphoreType.DMA((NBUF,)),
    ],
)(_sc_body)


@jax.jit
def kernel(x, edge_index, edge_vals, W, b):
    h2 = _matmul_split(x, W)
    src = edge_index[0].reshape(NS, J, C)
    dst = edge_index[1].reshape(NS, J, C)
    vals = edge_vals.reshape(NS, J, C)
    return _sc_kernel(h2, src, dst, vals, b)
